# baseline (speedup 1.0000x reference)
import jax
import jax.numpy as jnp
from jax import lax
from jax.experimental import pallas as pl
from jax.experimental.pallas import tpu as pltpu
from jax.experimental.pallas import tpu_sc as plsc

NC = 2
NS = 16
L = 16
NW = NC * NS

B = 64
N = 80000
MAXLEN = 120000
ROWS_PER_W = B // NW
CHO = 16000
NCHUNK = N // CHO
PAD = 128
NSTEP = ROWS_PER_W * NCHUNK
NBUF = 3


def _sc_body(wav_hbm, noise_hbm, meta_hbm, out_hbm,
             nidx_v, widx_v, meta_v, nbuf, wbuf, obuf,
             gsem, wsem, osem):
    wid = lax.axis_index("s") * NC + lax.axis_index("c")

    pltpu.sync_copy(meta_hbm.at[pl.ds(L * wid, L)], nidx_v)
    pltpu.sync_copy(meta_hbm.at[pl.ds(NW * L + L + L * wid, L)], widx_v)
    pltpu.sync_copy(meta_hbm.at[pl.ds(NW * L, L)], meta_v)

    start_s = jnp.max(meta_v[...])
    c0 = pl.multiple_of(start_s & jnp.int32(-128), 128)
    r = start_s & jnp.int32(127)

    def start_in(k):
        p = k % NBUF
        t, c = divmod(k, NCHUNK)
        pltpu.async_copy(
            noise_hbm.at[nidx_v.at[pl.ds(8 * t, 1)],
                         pl.ds(c0 + c * CHO, CHO + PAD)],
            nbuf.at[p], gsem[p])
        pltpu.async_copy(
            wav_hbm.at[widx_v.at[pl.ds(8 * t, 1)], pl.ds(c * CHO, CHO)],
            wbuf.at[p], wsem[p])

    def wait_in(k):
        p = k % NBUF
        t, c = divmod(k, NCHUNK)
        pltpu.make_async_copy(
            noise_hbm.at[nidx_v.at[pl.ds(8 * t, 1)],
                         pl.ds(c0 + c * CHO, CHO + PAD)],
            nbuf.at[p], gsem[p]).wait()
        pltpu.make_async_copy(
            wav_hbm.at[widx_v.at[pl.ds(8 * t, 1)], pl.ds(c * CHO, CHO)],
            wbuf.at[p], wsem[p]).wait()

    def out_copy(k):
        p = k % 2
        t, c = divmod(k, NCHUNK)
        return pltpu.make_async_copy(
            obuf.at[p],
            out_hbm.at[widx_v.at[pl.ds(8 * t, 1)], pl.ds(c * CHO, CHO)],
            osem[p])

    start_in(0)
    start_in(1)
    for k in range(NSTEP):
        p = k % NBUF
        if k + 2 < NSTEP:
            start_in(k + 2)
        wait_in(k)
        if k >= 2:
            out_copy(k - 2).wait()

        po = k % 2

        @plsc.parallel_loop(0, CHO // L, 1, unroll=8)
        def body(j):
            nv = nbuf[p, 0, pl.ds(r + j * L, L)]
            wv = wbuf[p, 0, pl.ds(j * L, L)]
            res = jnp.float32(0.05) * nv + wv
            res = jnp.minimum(jnp.maximum(res, jnp.float32(-1.0)),
                              jnp.float32(1.0))
            obuf[po, 0, pl.ds(j * L, L)] = res

        out_copy(k).start()

    out_copy(NSTEP - 2).wait()
    out_copy(NSTEP - 1).wait()


@jax.jit
def _add_noise_sc(waveforms, noise_files, meta):
    mesh = plsc.VectorSubcoreMesh(
        core_axis_name="c", subcore_axis_name="s",
        num_cores=NC, num_subcores=NS,
    )
    fn = pl.kernel(
        _sc_body,
        out_type=jax.ShapeDtypeStruct((B, N), jnp.float32),
        mesh=mesh,
        scratch_types=[
            pltpu.VMEM((L,), jnp.int32),
            pltpu.VMEM((L,), jnp.int32),
            pltpu.VMEM((L,), jnp.int32),
            pltpu.VMEM((NBUF, 1, CHO + PAD), jnp.float32),
            pltpu.VMEM((NBUF, 1, CHO), jnp.float32),
            pltpu.VMEM((2, 1, CHO), jnp.float32),
            [pltpu.SemaphoreType.DMA] * NBUF,
            [pltpu.SemaphoreType.DMA] * NBUF,
            [pltpu.SemaphoreType.DMA] * 2,
        ],
        compiler_params=pltpu.CompilerParams(needs_layout_passes=False),
    )
    return fn(waveforms, noise_files, meta)


def kernel(waveforms, lengths, noise_files, random_index, start_index):
    del lengths
    ridx = random_index.astype(jnp.int32)
    start = start_index.astype(jnp.int32)
    k = jnp.arange(NW * L + L + NW * L, dtype=jnp.int32)
    w = k // L
    half = (k % L) // 8
    batch_row = jnp.where(k < NW * L, w, w - (NW + 1)) * 2 + half
    noise_row = ridx[jnp.clip(batch_row, 0, B - 1)]
    val = jnp.where(k < NW * L, noise_row,
                    jnp.where(k < NW * L + L, start, batch_row))
    meta = jnp.where((k % 8 == 0) | ((k >= NW * L) & (k < NW * L + L)),
                     val, 0)
    return _add_noise_sc(waveforms, noise_files, meta)


# speedup vs baseline: 1.9159x; 1.9159x over previous
import jax
import jax.numpy as jnp
from jax import lax
from jax.experimental import pallas as pl
from jax.experimental.pallas import tpu as pltpu
from jax.experimental.pallas import tpu_sc as plsc

NC = 2
NS = 16
L = 16
NW = NC * NS

B = 64
N = 80000
MAXLEN = 120000
ROWS_PER_W = B // NW
CHO = 16000
NCHUNK = N // CHO
PAD = 128
NSTEP = ROWS_PER_W * NCHUNK
NBUF = 3


def _sc_body(wav_hbm, noise_hbm, meta_hbm, out_hbm,
             nidx_v, widx_v, meta_v, nbuf, wbuf, obuf,
             gsem, wsem, osem):
    wid = lax.axis_index("s") * NC + lax.axis_index("c")

    pltpu.sync_copy(meta_hbm.at[pl.ds(L * wid, L)], nidx_v)
    pltpu.sync_copy(meta_hbm.at[pl.ds(NW * L + L + L * wid, L)], widx_v)
    pltpu.sync_copy(meta_hbm.at[pl.ds(NW * L, L)], meta_v)

    start_s = jnp.max(meta_v[...])
    c0 = pl.multiple_of(start_s & jnp.int32(-128), 128)
    r = start_s & jnp.int32(127)

    def start_in(k):
        p = k % NBUF
        t, c = divmod(k, NCHUNK)
        pltpu.async_copy(
            noise_hbm.at[nidx_v.at[pl.ds(8 * t, 1)],
                         pl.ds(c0 + c * CHO, CHO + PAD)],
            nbuf.at[p], gsem[p])
        pltpu.async_copy(
            wav_hbm.at[widx_v.at[pl.ds(8 * t, 1)], pl.ds(c * CHO, CHO)],
            wbuf.at[p], wsem[p])

    def wait_in(k):
        p = k % NBUF
        t, c = divmod(k, NCHUNK)
        pltpu.make_async_copy(
            noise_hbm.at[nidx_v.at[pl.ds(8 * t, 1)],
                         pl.ds(c0 + c * CHO, CHO + PAD)],
            nbuf.at[p], gsem[p]).wait()
        pltpu.make_async_copy(
            wav_hbm.at[widx_v.at[pl.ds(8 * t, 1)], pl.ds(c * CHO, CHO)],
            wbuf.at[p], wsem[p]).wait()

    def out_copy(k):
        p = k % 2
        t, c = divmod(k, NCHUNK)
        return pltpu.make_async_copy(
            obuf.at[p],
            out_hbm.at[widx_v.at[pl.ds(8 * t, 1)], pl.ds(c * CHO, CHO)],
            osem[p])

    start_in(0)
    start_in(1)
    for k in range(NSTEP):
        p = k % NBUF
        if k + 2 < NSTEP:
            start_in(k + 2)
        wait_in(k)
        if k >= 2:
            out_copy(k - 2).wait()

        po = k % 2

        @plsc.parallel_loop(0, CHO // L, 1, unroll=8)
        def body(j):
            nv = nbuf[p, 0, pl.ds(r + j * L, L)]
            wv = wbuf[p, 0, pl.ds(j * L, L)]
            res = jnp.float32(0.05) * nv + wv
            res = jnp.minimum(jnp.maximum(res, jnp.float32(-1.0)),
                              jnp.float32(1.0))
            obuf[po, 0, pl.ds(j * L, L)] = res

        out_copy(k).start()

    out_copy(NSTEP - 2).wait()
    out_copy(NSTEP - 1).wait()


@jax.jit
def _add_noise_sc(waveforms, noise_files, meta):
    mesh = plsc.VectorSubcoreMesh(
        core_axis_name="c", subcore_axis_name="s",
        num_cores=NC, num_subcores=NS,
    )
    fn = pl.kernel(
        _sc_body,
        out_type=jax.ShapeDtypeStruct((B, N), jnp.float32),
        mesh=mesh,
        scratch_types=[
            pltpu.VMEM((L,), jnp.int32),
            pltpu.VMEM((L,), jnp.int32),
            pltpu.VMEM((L,), jnp.int32),
            pltpu.VMEM((NBUF, 1, CHO + PAD), jnp.float32),
            pltpu.VMEM((NBUF, 1, CHO), jnp.float32),
            pltpu.VMEM((2, 1, CHO), jnp.float32),
            [pltpu.SemaphoreType.DMA] * NBUF,
            [pltpu.SemaphoreType.DMA] * NBUF,
            [pltpu.SemaphoreType.DMA] * 2,
        ],
        compiler_params=pltpu.CompilerParams(needs_layout_passes=False),
    )
    return fn(waveforms, noise_files, meta)


def kernel(waveforms, lengths, noise_files, random_index, start_index):
    del lengths
    ridx = random_index.astype(jnp.int32)
    start = start_index.astype(jnp.int32)
    npairs = jnp.zeros((NW, L), jnp.int32)
    npairs = npairs.at[:, 0].set(ridx[0::2]).at[:, 8].set(ridx[1::2])
    wi = jnp.arange(NW, dtype=jnp.int32) * 2
    wpairs = jnp.zeros((NW, L), jnp.int32)
    wpairs = wpairs.at[:, 0].set(wi).at[:, 8].set(wi + 1)
    meta = jnp.concatenate(
        [npairs.reshape(-1), jnp.full((L,), start, jnp.int32),
         wpairs.reshape(-1)])
    return _add_noise_sc(waveforms, noise_files, meta)


# speedup vs baseline: 1.9838x; 1.0354x over previous
import jax
import jax.numpy as jnp
from jax import lax
from jax.experimental import pallas as pl
from jax.experimental.pallas import tpu as pltpu
from jax.experimental.pallas import tpu_sc as plsc

NC = 2
NS = 16
L = 16
NW = NC * NS

B = 64
N = 80000
MAXLEN = 120000
ROWS_PER_W = B // NW
CHO = 16000
NCHUNK = N // CHO
PAD = 128
NSTEP = ROWS_PER_W * NCHUNK
NBUF = 3


def _sc_body(wav_hbm, noise_hbm, ridx_hbm, start_hbm, out_hbm,
             nidx_v, widx_v, meta_v, ridx_v, nbuf, wbuf, obuf,
             gsem, wsem, osem):
    wid = lax.axis_index("s") * NC + lax.axis_index("c")
    lane = lax.iota(jnp.int32, L)

    meta_v[...] = jnp.zeros((L,), jnp.int32)
    pltpu.sync_copy(start_hbm, meta_v.at[pl.ds(0, 1)])
    pltpu.sync_copy(ridx_hbm, ridx_v)
    pair = ridx_v[pl.ds(2 * wid, L)]
    sel = jnp.where(lane < 8, 0, 1)
    nidx_v[...] = jnp.take_along_axis(pair, sel, axis=0)
    widx_v[...] = 2 * wid + jnp.where(lane < 8, 0, 1)

    start_s = jnp.max(meta_v[...])
    c0 = pl.multiple_of(start_s & jnp.int32(-128), 128)
    r = start_s & jnp.int32(127)

    def start_in(k):
        p = k % NBUF
        t, c = divmod(k, NCHUNK)
        pltpu.async_copy(
            noise_hbm.at[nidx_v.at[pl.ds(8 * t, 1)],
                         pl.ds(c0 + c * CHO, CHO + PAD)],
            nbuf.at[p], gsem[p])
        pltpu.async_copy(
            wav_hbm.at[widx_v.at[pl.ds(8 * t, 1)], pl.ds(c * CHO, CHO)],
            wbuf.at[p], wsem[p])

    def wait_in(k):
        p = k % NBUF
        t, c = divmod(k, NCHUNK)
        pltpu.make_async_copy(
            noise_hbm.at[nidx_v.at[pl.ds(8 * t, 1)],
                         pl.ds(c0 + c * CHO, CHO + PAD)],
            nbuf.at[p], gsem[p]).wait()
        pltpu.make_async_copy(
            wav_hbm.at[widx_v.at[pl.ds(8 * t, 1)], pl.ds(c * CHO, CHO)],
            wbuf.at[p], wsem[p]).wait()

    def out_copy(k):
        p = k % 2
        t, c = divmod(k, NCHUNK)
        return pltpu.make_async_copy(
            obuf.at[p],
            out_hbm.at[widx_v.at[pl.ds(8 * t, 1)], pl.ds(c * CHO, CHO)],
            osem[p])

    start_in(0)
    start_in(1)
    for k in range(NSTEP):
        p = k % NBUF
        if k + 2 < NSTEP:
            start_in(k + 2)
        wait_in(k)
        if k >= 2:
            out_copy(k - 2).wait()

        po = k % 2

        @plsc.parallel_loop(0, CHO // L, 1, unroll=8)
        def body(j):
            nv = nbuf[p, 0, pl.ds(r + j * L, L)]
            wv = wbuf[p, 0, pl.ds(j * L, L)]
            res = jnp.float32(0.05) * nv + wv
            res = jnp.minimum(jnp.maximum(res, jnp.float32(-1.0)),
                              jnp.float32(1.0))
            obuf[po, 0, pl.ds(j * L, L)] = res

        out_copy(k).start()

    out_copy(NSTEP - 2).wait()
    out_copy(NSTEP - 1).wait()


@jax.jit
def _add_noise_sc(waveforms, noise_files, ridx, start_arr):
    mesh = plsc.VectorSubcoreMesh(
        core_axis_name="c", subcore_axis_name="s",
        num_cores=NC, num_subcores=NS,
    )
    fn = pl.kernel(
        _sc_body,
        out_type=jax.ShapeDtypeStruct((B, N), jnp.float32),
        mesh=mesh,
        scratch_types=[
            pltpu.VMEM((L,), jnp.int32),
            pltpu.VMEM((L,), jnp.int32),
            pltpu.VMEM((L,), jnp.int32),
            pltpu.VMEM((B + L,), jnp.int32),
            pltpu.VMEM((NBUF, 1, CHO + PAD), jnp.float32),
            pltpu.VMEM((NBUF, 1, CHO), jnp.float32),
            pltpu.VMEM((2, 1, CHO), jnp.float32),
            [pltpu.SemaphoreType.DMA] * NBUF,
            [pltpu.SemaphoreType.DMA] * NBUF,
            [pltpu.SemaphoreType.DMA] * 2,
        ],
        compiler_params=pltpu.CompilerParams(needs_layout_passes=False),
    )
    return fn(waveforms, noise_files, ridx, start_arr)


def kernel(waveforms, lengths, noise_files, random_index, start_index):
    del lengths
    ridx = jnp.pad(random_index.astype(jnp.int32), (0, L))
    start_arr = start_index.astype(jnp.int32).reshape(1)
    return _add_noise_sc(waveforms, noise_files, ridx, start_arr)


# speedup vs baseline: 2.0923x; 1.0547x over previous
import jax
import jax.numpy as jnp
from jax import lax
from jax.experimental import pallas as pl
from jax.experimental.pallas import tpu as pltpu
from jax.experimental.pallas import tpu_sc as plsc

NC = 2
NS = 16
L = 16
NW = NC * NS

B = 64
N = 80000
MAXLEN = 120000
ROWS_PER_W = B // NW
CHO = 16000
NCHUNK = N // CHO
PAD = 128
NSTEP = ROWS_PER_W * NCHUNK
NBUF = 3


def _sc_body(wav_hbm, noise_hbm, ridx_hbm, start_hbm, out_hbm,
             nidx_v, widx_v, meta_v, ridx_v, nbuf, wbuf, obuf,
             gsem, wsem, osem):
    wid = lax.axis_index("s") * NC + lax.axis_index("c")
    lane = lax.iota(jnp.int32, L)

    meta_v[...] = jnp.zeros((L,), jnp.int32)
    pltpu.sync_copy(start_hbm, meta_v.at[pl.ds(0, 1)])
    pltpu.sync_copy(ridx_hbm, ridx_v)
    pair = ridx_v[pl.ds(2 * wid, L)]
    sel = jnp.where(lane < 8, 0, 1)
    nidx_v[...] = jnp.take_along_axis(pair, sel, axis=0)
    widx_v[...] = 2 * wid + jnp.where(lane < 8, 0, 1)

    start_s = jnp.max(meta_v[...])
    c0 = pl.multiple_of(start_s & jnp.int32(-128), 128)
    r = start_s & jnp.int32(127)

    def start_in(k):
        p = lax.rem(k, NBUF)
        t = lax.div(k, NCHUNK)
        c = lax.rem(k, NCHUNK)
        pltpu.async_copy(
            noise_hbm.at[nidx_v.at[pl.ds(8 * t, 1)],
                         pl.ds(c0 + c * CHO, CHO + PAD)],
            nbuf.at[p], gsem.at[p])
        pltpu.async_copy(
            wav_hbm.at[widx_v.at[pl.ds(8 * t, 1)], pl.ds(c * CHO, CHO)],
            wbuf.at[p], wsem.at[p])

    def wait_in(k):
        p = lax.rem(k, NBUF)
        t = lax.div(k, NCHUNK)
        c = lax.rem(k, NCHUNK)
        pltpu.make_async_copy(
            noise_hbm.at[nidx_v.at[pl.ds(8 * t, 1)],
                         pl.ds(c0 + c * CHO, CHO + PAD)],
            nbuf.at[p], gsem.at[p]).wait()
        pltpu.make_async_copy(
            wav_hbm.at[widx_v.at[pl.ds(8 * t, 1)], pl.ds(c * CHO, CHO)],
            wbuf.at[p], wsem.at[p]).wait()

    def out_copy(k):
        p = lax.rem(k, 2)
        t = lax.div(k, NCHUNK)
        c = lax.rem(k, NCHUNK)
        return pltpu.make_async_copy(
            obuf.at[p],
            out_hbm.at[widx_v.at[pl.ds(8 * t, 1)], pl.ds(c * CHO, CHO)],
            osem.at[p])

    start_in(jnp.int32(0))
    start_in(jnp.int32(1))

    def step(k, _):
        p = lax.rem(k, NBUF)
        po = lax.rem(k, 2)

        @pl.when(k + 2 < NSTEP)
        def _():
            start_in(k + 2)

        wait_in(k)

        @pl.when(k >= 2)
        def _():
            out_copy(k - 2).wait()

        @plsc.parallel_loop(0, CHO // L, 1, unroll=8)
        def body(j):
            nv = nbuf[p, 0, pl.ds(r + j * L, L)]
            wv = wbuf[p, 0, pl.ds(j * L, L)]
            res = jnp.float32(0.05) * nv + wv
            res = jnp.minimum(jnp.maximum(res, jnp.float32(-1.0)),
                              jnp.float32(1.0))
            obuf[po, 0, pl.ds(j * L, L)] = res

        out_copy(k).start()
        return _

    lax.fori_loop(0, NSTEP, step, None)
    out_copy(jnp.int32(NSTEP - 2)).wait()
    out_copy(jnp.int32(NSTEP - 1)).wait()


@jax.jit
def _add_noise_sc(waveforms, noise_files, ridx, start_arr):
    mesh = plsc.VectorSubcoreMesh(
        core_axis_name="c", subcore_axis_name="s",
        num_cores=NC, num_subcores=NS,
    )
    fn = pl.kernel(
        _sc_body,
        out_type=jax.ShapeDtypeStruct((B, N), jnp.float32),
        mesh=mesh,
        scratch_types=[
            pltpu.VMEM((L,), jnp.int32),
            pltpu.VMEM((L,), jnp.int32),
            pltpu.VMEM((L,), jnp.int32),
            pltpu.VMEM((B + L,), jnp.int32),
            pltpu.VMEM((NBUF, 1, CHO + PAD), jnp.float32),
            pltpu.VMEM((NBUF, 1, CHO), jnp.float32),
            pltpu.VMEM((2, 1, CHO), jnp.float32),
            pltpu.SemaphoreType.DMA((NBUF,)),
            pltpu.SemaphoreType.DMA((NBUF,)),
            pltpu.SemaphoreType.DMA((2,)),
        ],
        compiler_params=pltpu.CompilerParams(needs_layout_passes=False),
    )
    return fn(waveforms, noise_files, ridx, start_arr)


def kernel(waveforms, lengths, noise_files, random_index, start_index):
    del lengths
    ridx = jnp.pad(random_index.astype(jnp.int32), (0, L))
    start_arr = start_index.astype(jnp.int32).reshape(1)
    return _add_noise_sc(waveforms, noise_files, ridx, start_arr)


# speedup vs baseline: 2.1498x; 1.0275x over previous
import jax
import jax.numpy as jnp
from jax import lax
from jax.experimental import pallas as pl
from jax.experimental.pallas import tpu as pltpu
from jax.experimental.pallas import tpu_sc as plsc

NC = 2
NS = 16
L = 16
NW = NC * NS

B = 64
N = 80000
MAXLEN = 120000
ROWS_PER_W = B // NW
CHO = 16000
NCHUNK = N // CHO
PAD = 128
NSTEP = ROWS_PER_W * NCHUNK
NBUF = 3


def _sc_body(wav_hbm, noise_hbm, ridx_hbm, start_hbm, out_hbm,
             nidx_v, widx_v, meta_v, ridx_v, nbuf, wbuf, obuf,
             gsem, wsem, osem):
    wid = lax.axis_index("s") * NC + lax.axis_index("c")
    lane = lax.iota(jnp.int32, L)

    meta_v[...] = jnp.zeros((L,), jnp.int32)
    pltpu.sync_copy(start_hbm, meta_v.at[pl.ds(0, 1)])
    pltpu.sync_copy(ridx_hbm, ridx_v.at[pl.ds(0, B)])
    pair = ridx_v[pl.ds(2 * wid, L)]
    sel = jnp.where(lane < 8, 0, 1)
    nidx_v[...] = jnp.take_along_axis(pair, sel, axis=0)
    widx_v[...] = 2 * wid + jnp.where(lane < 8, 0, 1)

    start_s = jnp.max(meta_v[...])
    c0 = pl.multiple_of(start_s & jnp.int32(-128), 128)
    r = start_s & jnp.int32(127)

    def start_in(k):
        p = lax.rem(k, NBUF)
        t = lax.div(k, NCHUNK)
        c = lax.rem(k, NCHUNK)
        pltpu.async_copy(
            noise_hbm.at[nidx_v.at[pl.ds(8 * t, 1)],
                         pl.ds(c0 + c * CHO, CHO + PAD)],
            nbuf.at[p], gsem.at[p])
        pltpu.async_copy(
            wav_hbm.at[widx_v.at[pl.ds(8 * t, 1)], pl.ds(c * CHO, CHO)],
            wbuf.at[p], wsem.at[p])

    def wait_in(k):
        p = lax.rem(k, NBUF)
        t = lax.div(k, NCHUNK)
        c = lax.rem(k, NCHUNK)
        pltpu.make_async_copy(
            noise_hbm.at[nidx_v.at[pl.ds(8 * t, 1)],
                         pl.ds(c0 + c * CHO, CHO + PAD)],
            nbuf.at[p], gsem.at[p]).wait()
        pltpu.make_async_copy(
            wav_hbm.at[widx_v.at[pl.ds(8 * t, 1)], pl.ds(c * CHO, CHO)],
            wbuf.at[p], wsem.at[p]).wait()

    def out_copy(k):
        p = lax.rem(k, 2)
        t = lax.div(k, NCHUNK)
        c = lax.rem(k, NCHUNK)
        return pltpu.make_async_copy(
            obuf.at[p],
            out_hbm.at[widx_v.at[pl.ds(8 * t, 1)], pl.ds(c * CHO, CHO)],
            osem.at[p])

    start_in(jnp.int32(0))
    start_in(jnp.int32(1))

    def step(k, _):
        p = lax.rem(k, NBUF)
        po = lax.rem(k, 2)

        @pl.when(k + 2 < NSTEP)
        def _():
            start_in(k + 2)

        wait_in(k)

        @pl.when(k >= 2)
        def _():
            out_copy(k - 2).wait()

        @plsc.parallel_loop(0, CHO // L, 1, unroll=8)
        def body(j):
            nv = nbuf[p, 0, pl.ds(r + j * L, L)]
            wv = wbuf[p, 0, pl.ds(j * L, L)]
            res = jnp.float32(0.05) * nv + wv
            res = jnp.minimum(jnp.maximum(res, jnp.float32(-1.0)),
                              jnp.float32(1.0))
            obuf[po, 0, pl.ds(j * L, L)] = res

        out_copy(k).start()
        return _

    lax.fori_loop(0, NSTEP, step, None)
    out_copy(jnp.int32(NSTEP - 2)).wait()
    out_copy(jnp.int32(NSTEP - 1)).wait()


@jax.jit
def _add_noise_sc(waveforms, noise_files, ridx, start_arr):
    mesh = plsc.VectorSubcoreMesh(
        core_axis_name="c", subcore_axis_name="s",
        num_cores=NC, num_subcores=NS,
    )
    fn = pl.kernel(
        _sc_body,
        out_type=jax.ShapeDtypeStruct((B, N), jnp.float32),
        mesh=mesh,
        scratch_types=[
            pltpu.VMEM((L,), jnp.int32),
            pltpu.VMEM((L,), jnp.int32),
            pltpu.VMEM((L,), jnp.int32),
            pltpu.VMEM((B + L,), jnp.int32),
            pltpu.VMEM((NBUF, 1, CHO + PAD), jnp.float32),
            pltpu.VMEM((NBUF, 1, CHO), jnp.float32),
            pltpu.VMEM((2, 1, CHO), jnp.float32),
            pltpu.SemaphoreType.DMA((NBUF,)),
            pltpu.SemaphoreType.DMA((NBUF,)),
            pltpu.SemaphoreType.DMA((2,)),
        ],
        compiler_params=pltpu.CompilerParams(needs_layout_passes=False),
    )
    return fn(waveforms, noise_files, ridx, start_arr)


def kernel(waveforms, lengths, noise_files, random_index, start_index):
    del lengths
    ridx = random_index.astype(jnp.int32)
    start_arr = start_index.astype(jnp.int32).reshape(1)
    return _add_noise_sc(waveforms, noise_files, ridx, start_arr)
